# baseline (speedup 1.0000x reference)
import functools

import jax
import jax.numpy as jnp
from jax import lax
from jax.experimental import pallas as pl

B = 2
S = 4096
Q_DIM = 2048
KV_RANK = 512
IDX_DIM = 128
TOPK = 2048
TEMPERATURE = 11.3137085

BQ = 256

_BIG = 1e30
_STEP = 1e26

_PREC = lax.Precision.DEFAULT


def _roll(x, shift):
    shift %= x.shape[-1]
    if shift == 0:
        return x
    return jnp.concatenate([x[..., -shift:], x[..., :-shift]], axis=-1)


def _bitonic_sort_asc(key, pay):
    n = key.shape[-1]
    iota = lax.broadcasted_iota(jnp.int32, key.shape, key.ndim - 1)
    k = 2
    while k <= n:
        j = k // 2
        asc = (iota & k) == 0
        while j >= 1:
            is_lo = (iota & j) == 0
            pk_dn, pk_up = _roll(key, -j), _roll(key, j)
            pp_dn, pp_up = _roll(pay, -j), _roll(pay, j)
            partner_k = jnp.where(is_lo, pk_dn, pk_up)
            partner_p = jnp.where(is_lo, pp_dn, pp_up)
            want_min = is_lo == asc
            new_key = jnp.where(want_min,
                                jnp.minimum(key, partner_k),
                                jnp.maximum(key, partner_k))
            swap = (key <= partner_k) != want_min
            new_pay = jnp.where(swap, partner_p, pay)
            key, pay = new_key, new_pay
            j //= 2
        k *= 2
    return key, pay


def _topk_body(qidx_ref, kidx_ref, idx_ref, score_ref):
    qi = pl.program_id(1)
    scores = lax.dot_general(qidx_ref[0], kidx_ref[0],
                             (((1,), (1,)), ((), ())),
                             preferred_element_type=jnp.float32,
                             precision=_PREC) / TEMPERATURE
    rows = qi * BQ + lax.broadcasted_iota(jnp.int32, (BQ, S), 0)
    cols = lax.broadcasted_iota(jnp.int32, (BQ, S), 1)
    valid = cols <= rows
    key = jnp.where(valid, -scores, _BIG + cols.astype(jnp.float32) * _STEP)
    key, pay = _bitonic_sort_asc(key, cols)
    s_top = -key[:, :TOPK]
    m = s_top[:, 0:1]
    p = jnp.exp(s_top - m)
    score_ref[0] = p / jnp.sum(p, axis=-1, keepdims=True)
    idx_ref[0] = pay[:, :TOPK]


def kernel(q_compressed, kv_compressed, Wq, Wk, k_gamma):
    q_idx = q_compressed @ Wq
    k = kv_compressed @ Wk
    var = jnp.mean(k * k, axis=-1, keepdims=True)
    k_idx = k * lax.rsqrt(var + 1e-6) * k_gamma

    idx_out, score_out = pl.pallas_call(
        _topk_body,
        grid=(B, S // BQ),
        in_specs=[
            pl.BlockSpec((1, BQ, IDX_DIM), lambda b, i: (b, i, 0)),
            pl.BlockSpec((1, S, IDX_DIM), lambda b, i: (b, 0, 0)),
        ],
        out_specs=[
            pl.BlockSpec((1, BQ, TOPK), lambda b, i: (b, i, 0)),
            pl.BlockSpec((1, BQ, TOPK), lambda b, i: (b, i, 0)),
        ],
        out_shape=[
            jax.ShapeDtypeStruct((B, S, TOPK), jnp.int32),
            jax.ShapeDtypeStruct((B, S, TOPK), jnp.float32),
        ],
    )(q_idx, k_idx)
    return idx_out, score_out


# speedup vs baseline: 1.3393x; 1.3393x over previous
import functools

import jax
import jax.numpy as jnp
from jax import lax
from jax.experimental import pallas as pl

B = 2
S = 4096
Q_DIM = 2048
KV_RANK = 512
IDX_DIM = 128
TOPK = 2048
TEMPERATURE = 11.3137085

BQ = 256

_BIG = 1e30
_STEP = 1e26

_PREC = lax.Precision.DEFAULT


def _roll(x, shift):
    shift %= x.shape[-1]
    if shift == 0:
        return x
    return jnp.concatenate([x[..., -shift:], x[..., :-shift]], axis=-1)


def _bitonic_sort_asc(key, pay):
    n = key.shape[-1]
    iota = lax.broadcasted_iota(jnp.int32, key.shape, key.ndim - 1)
    k = 2
    while k <= n:
        j = k // 2
        asc = (iota & k) == 0
        while j >= 1:
            is_lo = (iota & j) == 0
            pk_dn, pk_up = _roll(key, -j), _roll(key, j)
            pp_dn, pp_up = _roll(pay, -j), _roll(pay, j)
            partner_k = jnp.where(is_lo, pk_dn, pk_up)
            partner_p = jnp.where(is_lo, pp_dn, pp_up)
            want_min = is_lo == asc
            new_key = jnp.where(want_min,
                                jnp.minimum(key, partner_k),
                                jnp.maximum(key, partner_k))
            swap = (key <= partner_k) != want_min
            new_pay = jnp.where(swap, partner_p, pay)
            key, pay = new_key, new_pay
            j //= 2
        k *= 2
    return key, pay


def _topk_body(qidx_ref, kidx_ref, idx_ref, score_ref, *, width, row0):
    qi = pl.program_id(1)
    scores = lax.dot_general(qidx_ref[0], kidx_ref[0],
                             (((1,), (1,)), ((), ())),
                             preferred_element_type=jnp.float32,
                             precision=_PREC) / TEMPERATURE
    rows = row0 + qi * BQ + lax.broadcasted_iota(jnp.int32, (BQ, width), 0)
    cols = lax.broadcasted_iota(jnp.int32, (BQ, width), 1)
    valid = cols <= rows
    key = jnp.where(valid, -scores, _BIG + cols.astype(jnp.float32) * _STEP)
    key, pay = _bitonic_sort_asc(key, cols)
    s_top = -key[:, :TOPK]
    m = s_top[:, 0:1]
    p = jnp.exp(s_top - m)
    score_ref[0] = p / jnp.sum(p, axis=-1, keepdims=True)
    idx_ref[0] = pay[:, :TOPK]


def kernel(q_compressed, kv_compressed, Wq, Wk, k_gamma):
    q_idx = q_compressed @ Wq
    k = kv_compressed @ Wk
    var = jnp.mean(k * k, axis=-1, keepdims=True)
    k_idx = k * lax.rsqrt(var + 1e-6) * k_gamma

    H = S // 2
    idx_lo, score_lo = pl.pallas_call(
        functools.partial(_topk_body, width=H, row0=0),
        grid=(B, H // BQ),
        in_specs=[
            pl.BlockSpec((1, BQ, IDX_DIM), lambda b, i: (b, i, 0)),
            pl.BlockSpec((1, H, IDX_DIM), lambda b, i: (b, 0, 0)),
        ],
        out_specs=[
            pl.BlockSpec((1, BQ, TOPK), lambda b, i: (b, i, 0)),
            pl.BlockSpec((1, BQ, TOPK), lambda b, i: (b, i, 0)),
        ],
        out_shape=[
            jax.ShapeDtypeStruct((B, H, TOPK), jnp.int32),
            jax.ShapeDtypeStruct((B, H, TOPK), jnp.float32),
        ],
    )(q_idx[:, :H], k_idx)

    nhi = H // BQ
    idx_hi, score_hi = pl.pallas_call(
        functools.partial(_topk_body, width=S, row0=H),
        grid=(B, nhi),
        in_specs=[
            pl.BlockSpec((1, BQ, IDX_DIM), lambda b, i: (b, i + nhi, 0)),
            pl.BlockSpec((1, S, IDX_DIM), lambda b, i: (b, 0, 0)),
        ],
        out_specs=[
            pl.BlockSpec((1, BQ, TOPK), lambda b, i: (b, i, 0)),
            pl.BlockSpec((1, BQ, TOPK), lambda b, i: (b, i, 0)),
        ],
        out_shape=[
            jax.ShapeDtypeStruct((B, H, TOPK), jnp.int32),
            jax.ShapeDtypeStruct((B, H, TOPK), jnp.float32),
        ],
    )(q_idx, k_idx)
    idx_out = jnp.concatenate([idx_lo, idx_hi], axis=1)
    score_out = jnp.concatenate([score_lo, score_hi], axis=1)
    return idx_out, score_out
